# Initial kernel scaffold; baseline (speedup 1.0000x reference)
import jax
import jax.numpy as jnp
from jax.experimental import pallas as pl


def kernel(node_ids, edge_index, edge_dist, poi_table, delta_dis_embs, w_gate, b_gate):
    raise NotImplementedError("write your pallas kernel here")


# speedup vs baseline: 2.5543x; 2.5543x over previous
import functools

import jax
import jax.numpy as jnp
from jax import lax
from jax.experimental import pallas as pl
from jax.experimental.pallas import tpu as pltpu
from jax.experimental.pallas import tpu_sc as plsc

N = 50000
D = 64
E = 800000
NC = 2
NS = 16
NW = NC * NS

TILE_ROWS = 1568
NPAD = NW * TILE_ROWS
HALF = NPAD // NC
AGG_ROWS = HALF + 16
RC = 112
NRC = TILE_ROWS // RC
EC = 128
EDGES_PER_TILE = 50176
EPAD = NS * EDGES_PER_TILE
NECH = EDGES_PER_TILE // EC

_mesh = plsc.VectorSubcoreMesh(core_axis_name="c", subcore_axis_name="s")
_params = pltpu.CompilerParams(
    use_tc_tiling_on_sc=False, needs_layout_passes=False)


def _zero_rows(ref, nrows):
    z = jnp.zeros((16,), jnp.float32)

    def body(r, _):
        for f in range(D // 16):
            ref[r, pl.ds(f * 16, 16)] = z
        return 0

    lax.fori_loop(0, nrows, body, 0)


def _fill_1d(ref, n, val):
    v = jnp.full((16,), val, jnp.float32)
    for i in range(n // 16):
        ref[pl.ds(i * 16, 16)] = v


def _local_dst(ib, loc, base):
    for i in range(EC // 16):
        v = ib[pl.ds(i * 16, 16)]
        l = v - base
        ok = (l >= 0) & (l < HALF)
        loc[pl.ds(i * 16, 16)] = jnp.where(ok, l, HALF)


def _k1_body(nid, poi, dstp, distp, dembs,
             x0, segd, deg,
             ibr, ib0, ib1, loc, m0, ones_v, dbuf, sem,
             seg_sh, deg_sh):
    c = lax.axis_index("c")
    s = lax.axis_index("s")
    wid = c * NS + s

    rbase = wid * TILE_ROWS

    def pass_a(j, _):
        b = rbase + j * RC
        pltpu.sync_copy(nid.at[pl.ds(b, RC)], ibr)
        pltpu.async_copy(poi.at[ibr], m0.at[pl.ds(0, RC), :], sem).wait()
        pltpu.sync_copy(m0.at[pl.ds(0, RC), :], x0.at[pl.ds(b, RC), :])
        return 0

    lax.fori_loop(0, NRC, pass_a, 0)

    _zero_rows(m0, RC)
    _fill_1d(dbuf, RC, 0.0)
    _fill_1d(ones_v, EC, 1.0)
    lbase = s * TILE_ROWS

    def zf(j, _):
        lb = lbase + j * RC
        pltpu.sync_copy(m0.at[pl.ds(0, RC), :], seg_sh.at[pl.ds(lb, RC), :])
        pltpu.sync_copy(dbuf, deg_sh.at[pl.ds(lb, RC)])
        return 0

    lax.fori_loop(0, NRC, zf, 0)
    plsc.subcore_barrier()

    ebase0 = s * EDGES_PER_TILE
    base = c * HALF

    def pass_b(j, _):
        eb = ebase0 + j * EC
        pltpu.sync_copy(dstp.at[pl.ds(eb, EC)], ib0)
        pltpu.sync_copy(distp.at[pl.ds(eb, EC)], ib1)
        cp = pltpu.async_copy(dembs.at[ib1], m0, sem)
        _local_dst(ib0, loc, base)
        cp.wait()
        pltpu.sync_copy(m0, seg_sh.at[loc], add=True)
        pltpu.sync_copy(ones_v, deg_sh.at[loc], add=True)
        return 0

    lax.fori_loop(0, NECH, pass_b, 0)
    plsc.subcore_barrier()

    def wb(j, _):
        lb = lbase + j * RC
        g = base + lb
        pltpu.sync_copy(seg_sh.at[pl.ds(lb, RC), :], m0.at[pl.ds(0, RC), :])
        pltpu.sync_copy(m0.at[pl.ds(0, RC), :], segd.at[pl.ds(g, RC), :])
        pltpu.sync_copy(deg_sh.at[pl.ds(lb, RC)], dbuf)
        for i in range(RC // 16):
            dbuf[pl.ds(i * 16, 16)] = jnp.maximum(dbuf[pl.ds(i * 16, 16)], 1.0)
        pltpu.sync_copy(dbuf, deg.at[pl.ds(g, RC)])
        return 0

    lax.fori_loop(0, NRC, wb, 0)


def _layer_body(final, xl, segd, deg, srcp, dstp, wb_arr,
                y,
                ib0, ib1, loc, m0, m1, dbuf, rbuf, wv, sem,
                agg_sh):
    c = lax.axis_index("c")
    s = lax.axis_index("s")
    base = c * HALF
    lbase = s * TILE_ROWS

    if final:
        pltpu.sync_copy(wb_arr, wv)

    def p1(j, _):
        lb = lbase + j * RC
        g = base + lb
        pltpu.sync_copy(segd.at[pl.ds(g, RC), :], m0.at[pl.ds(0, RC), :])
        pltpu.sync_copy(m0.at[pl.ds(0, RC), :], agg_sh.at[pl.ds(lb, RC), :])
        return 0

    lax.fori_loop(0, NRC, p1, 0)
    plsc.subcore_barrier()

    ebase0 = s * EDGES_PER_TILE

    def p2(j, _):
        eb = ebase0 + j * EC
        pltpu.sync_copy(srcp.at[pl.ds(eb, EC)], ib0)
        pltpu.sync_copy(dstp.at[pl.ds(eb, EC)], ib1)
        cp = pltpu.async_copy(xl.at[ib0], m0, sem)
        _local_dst(ib1, loc, base)
        cp.wait()
        pltpu.sync_copy(m0, agg_sh.at[loc], add=True)
        return 0

    lax.fori_loop(0, NECH, p2, 0)
    plsc.subcore_barrier()

    def p3(j, _):
        lb = lbase + j * RC
        g = base + lb
        pltpu.sync_copy(agg_sh.at[pl.ds(lb, RC), :], m0.at[pl.ds(0, RC), :])
        pltpu.sync_copy(xl.at[pl.ds(g, RC), :], m1.at[pl.ds(0, RC), :])
        pltpu.sync_copy(deg.at[pl.ds(g, RC)], dbuf)
        for i in range(RC // 16):
            rbuf[pl.ds(i * 16, 16)] = 1.0 / dbuf[pl.ds(i * 16, 16)]

        if final:
            def row(r, _):
                rv = rbuf[pl.ds(r, 16)][0]
                acc = jnp.zeros((16,), jnp.float32)
                for f in range(D // 16):
                    yv = m1[r, pl.ds(f * 16, 16)] + m0[r, pl.ds(f * 16, 16)] * rv
                    m0[r, pl.ds(f * 16, 16)] = yv
                    acc = acc + yv * wv[pl.ds(f * 16, 16)]
                z = jnp.sum(acc)
                zb = lax.broadcast_in_dim(z, (16,), ()) + wv[pl.ds(D, 16)]
                gv = 1.0 / (1.0 + jnp.exp(-zb))
                for f in range(D // 16):
                    m0[r, pl.ds(f * 16, 16)] = m0[r, pl.ds(f * 16, 16)] * gv
                return 0
        else:
            def row(r, _):
                rv = rbuf[pl.ds(r, 16)][0]
                for f in range(D // 16):
                    m0[r, pl.ds(f * 16, 16)] = (
                        m1[r, pl.ds(f * 16, 16)] + m0[r, pl.ds(f * 16, 16)] * rv
                    )
                return 0

        lax.fori_loop(0, RC, row, 0)
        pltpu.sync_copy(m0.at[pl.ds(0, RC), :], y.at[pl.ds(g, RC), :])
        return 0

    lax.fori_loop(0, NRC, p3, 0)


def _make_k1(num_pois):
    return pl.kernel(
        _k1_body,
        out_type=(
            jax.ShapeDtypeStruct((NPAD, D), jnp.float32),
            jax.ShapeDtypeStruct((NPAD, D), jnp.float32),
            jax.ShapeDtypeStruct((NPAD,), jnp.float32),
        ),
        mesh=_mesh,
        scratch_types=[
            pltpu.VMEM((RC,), jnp.int32),
            pltpu.VMEM((EC,), jnp.int32),
            pltpu.VMEM((EC,), jnp.int32),
            pltpu.VMEM((EC,), jnp.int32),
            pltpu.VMEM((EC, D), jnp.float32),
            pltpu.VMEM((EC,), jnp.float32),
            pltpu.VMEM((RC,), jnp.float32),
            pltpu.SemaphoreType.DMA,
            pltpu.VMEM_SHARED((AGG_ROWS, D), jnp.float32),
            pltpu.VMEM_SHARED((AGG_ROWS,), jnp.float32),
        ],
        compiler_params=_params,
        name="mgdc_k1",
    )


def _make_layer(final):
    return pl.kernel(
        functools.partial(_layer_body, final),
        out_type=jax.ShapeDtypeStruct((NPAD, D), jnp.float32),
        mesh=_mesh,
        scratch_types=[
            pltpu.VMEM((EC,), jnp.int32),
            pltpu.VMEM((EC,), jnp.int32),
            pltpu.VMEM((EC,), jnp.int32),
            pltpu.VMEM((EC, D), jnp.float32),
            pltpu.VMEM((EC, D), jnp.float32),
            pltpu.VMEM((RC,), jnp.float32),
            pltpu.VMEM((RC + 16,), jnp.float32),
            pltpu.VMEM((D + 16,), jnp.float32),
            pltpu.SemaphoreType.DMA,
            pltpu.VMEM_SHARED((AGG_ROWS, D), jnp.float32),
        ],
        compiler_params=_params,
        name="mgdc_layer",
    )


def kernel(node_ids, edge_index, edge_dist, poi_table, delta_dis_embs,
           w_gate, b_gate):
    node_ids = node_ids.astype(jnp.int32)
    src = edge_index[0].astype(jnp.int32)
    dst = edge_index[1].astype(jnp.int32)
    dist = edge_dist.astype(jnp.int32)
    poi = poi_table.astype(jnp.float32)
    dembs = delta_dis_embs.astype(jnp.float32)

    nid_p = jnp.concatenate([node_ids, jnp.zeros((NPAD - N,), jnp.int32)])
    srcp = jnp.concatenate([src, jnp.zeros((EPAD - E,), jnp.int32)])
    dstp = jnp.concatenate([dst, jnp.full((EPAD - E,), NPAD, jnp.int32)])
    distp = jnp.concatenate([dist, jnp.zeros((EPAD - E,), jnp.int32)])
    wb_arr = jnp.concatenate(
        [w_gate.reshape(D).astype(jnp.float32),
         jnp.full((16,), b_gate.reshape(-1)[0], jnp.float32)]
    )

    k1 = _make_k1(poi.shape[0])
    layer = _make_layer(final=False)
    layer_final = _make_layer(final=True)

    x0, segd, deg = k1(nid_p, poi, dstp, distp, dembs)
    x1 = layer(x0, segd, deg, srcp, dstp, wb_arr)
    out = layer_final(x1, segd, deg, srcp, dstp, wb_arr)
    return out[:N]


# speedup vs baseline: 3.0307x; 1.1865x over previous
import functools

import jax
import jax.numpy as jnp
from jax import lax
from jax.experimental import pallas as pl
from jax.experimental.pallas import tpu as pltpu
from jax.experimental.pallas import tpu_sc as plsc

N = 50000
D = 64
E = 800000
NC = 2
NS = 16
NW = NC * NS

TILE_ROWS = 1568
NPAD = NW * TILE_ROWS
HALF = NPAD // NC
AGG_ROWS = HALF + 16
RC = 112
NRC = TILE_ROWS // RC
EC = 128
EDGES_PER_TILE = 50176
EPAD = NS * EDGES_PER_TILE
NECH = EDGES_PER_TILE // EC

_mesh = plsc.VectorSubcoreMesh(core_axis_name="c", subcore_axis_name="s")
_params = pltpu.CompilerParams(
    use_tc_tiling_on_sc=False, needs_layout_passes=False)


def _zero_rows(ref, nrows):
    z = jnp.zeros((16,), jnp.float32)

    def body(r, _):
        for f in range(D // 16):
            ref[r, pl.ds(f * 16, 16)] = z
        return 0

    lax.fori_loop(0, nrows, body, 0)


def _fill_1d(ref, n, val):
    v = jnp.full((16,), val, jnp.float32)
    for i in range(n // 16):
        ref[pl.ds(i * 16, 16)] = v


def _local_dst(ib, loc, base):
    for i in range(EC // 16):
        v = ib[pl.ds(i * 16, 16)]
        l = v - base
        ok = (l >= 0) & (l < HALF)
        loc[pl.ds(i * 16, 16)] = jnp.where(ok, l, HALF)


def _k1_body(nid, poi, dstp, distp, dembs,
             x0, segd, deg,
             ibr, ib0a, ib0b, ib1a, ib1b, loca, locb, m0a, m0b,
             ones_v, dbuf, sem,
             si0, si1, sg0, sg1, ss0, ss1, sd0, sd1,
             seg_sh, deg_sh):
    c = lax.axis_index("c")
    s = lax.axis_index("s")
    wid = c * NS + s

    rbase = wid * TILE_ROWS

    def pass_a(j, _):
        b = rbase + j * RC
        pltpu.sync_copy(nid.at[pl.ds(b, RC)], ibr)
        pltpu.async_copy(poi.at[ibr], m0a.at[pl.ds(0, RC), :], sem).wait()
        pltpu.sync_copy(m0a.at[pl.ds(0, RC), :], x0.at[pl.ds(b, RC), :])
        return 0

    lax.fori_loop(0, NRC, pass_a, 0)

    _zero_rows(m0a, RC)
    _fill_1d(dbuf, RC, 0.0)
    _fill_1d(ones_v, EC, 1.0)
    lbase = s * TILE_ROWS

    def zf(j, _):
        lb = lbase + j * RC
        pltpu.sync_copy(m0a.at[pl.ds(0, RC), :], seg_sh.at[pl.ds(lb, RC), :])
        pltpu.sync_copy(dbuf, deg_sh.at[pl.ds(lb, RC)])
        return 0

    lax.fori_loop(0, NRC, zf, 0)
    plsc.subcore_barrier()

    ebase0 = s * EDGES_PER_TILE
    base = c * HALF
    ib0s, ib1s = (ib0a, ib0b), (ib1a, ib1b)
    locs, m0s = (loca, locb), (m0a, m0b)
    sis, sgs, sss, sds = (si0, si1), (sg0, sg1), (ss0, ss1), (sd0, sd1)

    def idx_issue(k, b):
        eb = ebase0 + k * EC
        pltpu.async_copy(dstp.at[pl.ds(eb, EC)], ib0s[b], sis[b])
        pltpu.async_copy(distp.at[pl.ds(eb, EC)], ib1s[b], sis[b])

    def idx_drain(b):
        pltpu.make_async_copy(dstp.at[pl.ds(0, EC)], ib0s[b], sis[b]).wait()
        pltpu.make_async_copy(distp.at[pl.ds(0, EC)], ib1s[b], sis[b]).wait()

    def row_drain(sem_ref, b):
        pltpu.make_async_copy(segd.at[pl.ds(0, EC), :], m0s[b], sem_ref).wait()

    def deg_drain(b):
        pltpu.make_async_copy(deg.at[pl.ds(0, EC)], ones_v, sds[b]).wait()

    idx_issue(0, 0)
    idx_drain(0)
    pltpu.async_copy(dembs.at[ib1s[0]], m0s[0], sgs[0])
    idx_issue(1, 1)

    def outer_b(g, _):
        for b in (0, 1):
            nb = 1 - b
            k = 2 * g + b

            @pl.when(k >= 1)
            def _():
                row_drain(sss[nb], nb)
                deg_drain(nb)

            @pl.when(k + 1 < NECH)
            def _():
                idx_drain(nb)
                pltpu.async_copy(dembs.at[ib1s[nb]], m0s[nb], sgs[nb])

            _local_dst(ib0s[b], locs[b], base)
            row_drain(sgs[b], b)

            @pl.when(k + 2 < NECH)
            def _():
                idx_issue(k + 2, b)

            pltpu.async_copy(m0s[b], seg_sh.at[locs[b]], sss[b], add=True)
            pltpu.async_copy(ones_v, deg_sh.at[locs[b]], sds[b], add=True)
        return 0

    lax.fori_loop(0, NECH // 2, outer_b, 0)
    row_drain(sss[1], 1)
    deg_drain(1)
    plsc.subcore_barrier()

    def wb(j, _):
        lb = lbase + j * RC
        g = base + lb
        pltpu.sync_copy(seg_sh.at[pl.ds(lb, RC), :], m0a.at[pl.ds(0, RC), :])
        pltpu.sync_copy(m0a.at[pl.ds(0, RC), :], segd.at[pl.ds(g, RC), :])
        pltpu.sync_copy(deg_sh.at[pl.ds(lb, RC)], dbuf)
        for i in range(RC // 16):
            dbuf[pl.ds(i * 16, 16)] = jnp.maximum(dbuf[pl.ds(i * 16, 16)], 1.0)
        pltpu.sync_copy(dbuf, deg.at[pl.ds(g, RC)])
        return 0

    lax.fori_loop(0, NRC, wb, 0)


def _layer_body(final, xl, segd, deg, srcp, dstp, wb_arr,
                y,
                ib0a, ib0b, ib1a, ib1b, loca, locb, m0a, m0b, m1,
                dbuf, rbuf, wv,
                si0, si1, sg0, sg1, ss0, ss1,
                agg_sh):
    c = lax.axis_index("c")
    s = lax.axis_index("s")
    base = c * HALF
    lbase = s * TILE_ROWS

    if final:
        pltpu.sync_copy(wb_arr, wv)

    def p1(j, _):
        lb = lbase + j * RC
        g = base + lb
        pltpu.sync_copy(segd.at[pl.ds(g, RC), :], m0a.at[pl.ds(0, RC), :])
        pltpu.sync_copy(m0a.at[pl.ds(0, RC), :], agg_sh.at[pl.ds(lb, RC), :])
        return 0

    lax.fori_loop(0, NRC, p1, 0)
    plsc.subcore_barrier()

    ebase0 = s * EDGES_PER_TILE
    ib0s, ib1s = (ib0a, ib0b), (ib1a, ib1b)
    locs, m0s = (loca, locb), (m0a, m0b)
    sis, sgs, sss = (si0, si1), (sg0, sg1), (ss0, ss1)

    def idx_issue(k, b):
        eb = ebase0 + k * EC
        pltpu.async_copy(srcp.at[pl.ds(eb, EC)], ib0s[b], sis[b])
        pltpu.async_copy(dstp.at[pl.ds(eb, EC)], ib1s[b], sis[b])

    def idx_drain(b):
        pltpu.make_async_copy(srcp.at[pl.ds(0, EC)], ib0s[b], sis[b]).wait()
        pltpu.make_async_copy(dstp.at[pl.ds(0, EC)], ib1s[b], sis[b]).wait()

    def row_drain(sem_ref, b):
        pltpu.make_async_copy(xl.at[pl.ds(0, EC), :], m0s[b], sem_ref).wait()

    idx_issue(0, 0)
    idx_drain(0)
    pltpu.async_copy(xl.at[ib0s[0]], m0s[0], sgs[0])
    idx_issue(1, 1)

    def p2(g, _):
        for b in (0, 1):
            nb = 1 - b
            k = 2 * g + b

            @pl.when(k >= 1)
            def _():
                row_drain(sss[nb], nb)

            @pl.when(k + 1 < NECH)
            def _():
                idx_drain(nb)
                pltpu.async_copy(xl.at[ib0s[nb]], m0s[nb], sgs[nb])

            _local_dst(ib1s[b], locs[b], base)
            row_drain(sgs[b], b)

            @pl.when(k + 2 < NECH)
            def _():
                idx_issue(k + 2, b)

            pltpu.async_copy(m0s[b], agg_sh.at[locs[b]], sss[b], add=True)
        return 0

    lax.fori_loop(0, NECH // 2, p2, 0)
    row_drain(sss[1], 1)
    plsc.subcore_barrier()

    def p3(j, _):
        lb = lbase + j * RC
        g = base + lb
        pltpu.sync_copy(agg_sh.at[pl.ds(lb, RC), :], m0a.at[pl.ds(0, RC), :])
        pltpu.sync_copy(xl.at[pl.ds(g, RC), :], m1.at[pl.ds(0, RC), :])
        pltpu.sync_copy(deg.at[pl.ds(g, RC)], dbuf)
        for i in range(RC // 16):
            rbuf[pl.ds(i * 16, 16)] = 1.0 / dbuf[pl.ds(i * 16, 16)]

        if final:
            def row(r, _):
                rv = rbuf[pl.ds(r, 16)][0]
                acc = jnp.zeros((16,), jnp.float32)
                for f in range(D // 16):
                    yv = m1[r, pl.ds(f * 16, 16)] + m0a[r, pl.ds(f * 16, 16)] * rv
                    m0a[r, pl.ds(f * 16, 16)] = yv
                    acc = acc + yv * wv[pl.ds(f * 16, 16)]
                z = jnp.sum(acc)
                zb = lax.broadcast_in_dim(z, (16,), ()) + wv[pl.ds(D, 16)]
                gv = 1.0 / (1.0 + jnp.exp(-zb))
                for f in range(D // 16):
                    m0a[r, pl.ds(f * 16, 16)] = m0a[r, pl.ds(f * 16, 16)] * gv
                return 0
        else:
            def row(r, _):
                rv = rbuf[pl.ds(r, 16)][0]
                for f in range(D // 16):
                    m0a[r, pl.ds(f * 16, 16)] = (
                        m1[r, pl.ds(f * 16, 16)] + m0a[r, pl.ds(f * 16, 16)] * rv
                    )
                return 0

        lax.fori_loop(0, RC, row, 0)
        pltpu.sync_copy(m0a.at[pl.ds(0, RC), :], y.at[pl.ds(g, RC), :])
        return 0

    lax.fori_loop(0, NRC, p3, 0)


def _make_k1(num_pois):
    return pl.kernel(
        _k1_body,
        out_type=(
            jax.ShapeDtypeStruct((NPAD, D), jnp.float32),
            jax.ShapeDtypeStruct((NPAD, D), jnp.float32),
            jax.ShapeDtypeStruct((NPAD,), jnp.float32),
        ),
        mesh=_mesh,
        scratch_types=[
            pltpu.VMEM((RC,), jnp.int32),
            pltpu.VMEM((EC,), jnp.int32),
            pltpu.VMEM((EC,), jnp.int32),
            pltpu.VMEM((EC,), jnp.int32),
            pltpu.VMEM((EC,), jnp.int32),
            pltpu.VMEM((EC,), jnp.int32),
            pltpu.VMEM((EC,), jnp.int32),
            pltpu.VMEM((EC, D), jnp.float32),
            pltpu.VMEM((EC, D), jnp.float32),
            pltpu.VMEM((EC,), jnp.float32),
            pltpu.VMEM((RC,), jnp.float32),
            pltpu.SemaphoreType.DMA,
            pltpu.SemaphoreType.DMA,
            pltpu.SemaphoreType.DMA,
            pltpu.SemaphoreType.DMA,
            pltpu.SemaphoreType.DMA,
            pltpu.SemaphoreType.DMA,
            pltpu.SemaphoreType.DMA,
            pltpu.SemaphoreType.DMA,
            pltpu.SemaphoreType.DMA,
            pltpu.VMEM_SHARED((AGG_ROWS, D), jnp.float32),
            pltpu.VMEM_SHARED((AGG_ROWS,), jnp.float32),
        ],
        compiler_params=_params,
        name="mgdc_k1",
    )


def _make_layer(final):
    return pl.kernel(
        functools.partial(_layer_body, final),
        out_type=jax.ShapeDtypeStruct((NPAD, D), jnp.float32),
        mesh=_mesh,
        scratch_types=[
            pltpu.VMEM((EC,), jnp.int32),
            pltpu.VMEM((EC,), jnp.int32),
            pltpu.VMEM((EC,), jnp.int32),
            pltpu.VMEM((EC,), jnp.int32),
            pltpu.VMEM((EC,), jnp.int32),
            pltpu.VMEM((EC,), jnp.int32),
            pltpu.VMEM((EC, D), jnp.float32),
            pltpu.VMEM((EC, D), jnp.float32),
            pltpu.VMEM((EC, D), jnp.float32),
            pltpu.VMEM((RC,), jnp.float32),
            pltpu.VMEM((RC + 16,), jnp.float32),
            pltpu.VMEM((D + 16,), jnp.float32),
            pltpu.SemaphoreType.DMA,
            pltpu.SemaphoreType.DMA,
            pltpu.SemaphoreType.DMA,
            pltpu.SemaphoreType.DMA,
            pltpu.SemaphoreType.DMA,
            pltpu.SemaphoreType.DMA,
            pltpu.VMEM_SHARED((AGG_ROWS, D), jnp.float32),
        ],
        compiler_params=_params,
        name="mgdc_layer",
    )


def kernel(node_ids, edge_index, edge_dist, poi_table, delta_dis_embs,
           w_gate, b_gate):
    node_ids = node_ids.astype(jnp.int32)
    src = edge_index[0].astype(jnp.int32)
    dst = edge_index[1].astype(jnp.int32)
    dist = edge_dist.astype(jnp.int32)
    poi = poi_table.astype(jnp.float32)
    dembs = delta_dis_embs.astype(jnp.float32)

    nid_p = jnp.concatenate([node_ids, jnp.zeros((NPAD - N,), jnp.int32)])
    srcp = jnp.concatenate([src, jnp.zeros((EPAD - E,), jnp.int32)])
    dstp = jnp.concatenate([dst, jnp.full((EPAD - E,), NPAD, jnp.int32)])
    distp = jnp.concatenate([dist, jnp.zeros((EPAD - E,), jnp.int32)])
    wb_arr = jnp.concatenate(
        [w_gate.reshape(D).astype(jnp.float32),
         jnp.full((16,), b_gate.reshape(-1)[0], jnp.float32)]
    )

    k1 = _make_k1(poi.shape[0])
    layer = _make_layer(final=False)
    layer_final = _make_layer(final=True)

    x0, segd, deg = k1(nid_p, poi, dstp, distp, dembs)
    x1 = layer(x0, segd, deg, srcp, dstp, wb_arr)
    out = layer_final(x1, segd, deg, srcp, dstp, wb_arr)
    return out[:N]


# speedup vs baseline: 3.6544x; 1.2058x over previous
import functools

import jax
import jax.numpy as jnp
from jax import lax
from jax.experimental import pallas as pl
from jax.experimental.pallas import tpu as pltpu
from jax.experimental.pallas import tpu_sc as plsc

N = 50000
D = 64
DA = 80
E = 800000
NC = 2
NS = 16
NW = NC * NS

TILE_ROWS = 1568
NPAD = NW * TILE_ROWS
HALF = NPAD // NC
AGG_ROWS = HALF + 16
RC = 112
NRC = TILE_ROWS // RC
EC = 128
EDGES_PER_TILE = E // NS
NECH = 390
ETAIL = EDGES_PER_TILE - NECH * EC
NTAIL_ROWS = 48

_mesh = plsc.VectorSubcoreMesh(core_axis_name="c", subcore_axis_name="s")
_params = pltpu.CompilerParams(
    use_tc_tiling_on_sc=False, needs_layout_passes=False)


def _zero_rows(ref, nrows, ncols):
    z = jnp.zeros((16,), jnp.float32)

    def body(r, _):
        for f in range(ncols // 16):
            ref[r, pl.ds(f * 16, 16)] = z
        return 0

    lax.fori_loop(0, nrows, body, 0)


def _fill_1d(ref, n, val):
    v = jnp.full((16,), val, jnp.float32)
    for i in range(n // 16):
        ref[pl.ds(i * 16, 16)] = v


def _local_dst(ib, loc, base, n):
    for i in range(n // 16):
        v = ib[pl.ds(i * 16, 16)]
        l = v - base
        ok = (l >= 0) & (l < HALF)
        loc[pl.ds(i * 16, 16)] = jnp.where(ok, l, HALF)


def _k1_body(nid, poi, dst_e, dist_e, dembs,
             x0, segd, deg,
             ibr, ibrt, ib0a, ib0b, ib1a, ib1b, loca, locb,
             ib0t, ib1t, loct, m0a, m0b, ma, ones_v, dbuf, sem,
             si0, si1, sg0, sg1, ss0, ss1, sd0, sd1,
             seg_sh, deg_sh):
    c = lax.axis_index("c")
    s = lax.axis_index("s")
    wid = c * NS + s

    rbase = wid * TILE_ROWS

    def pass_a(j, _):
        b = rbase + j * RC
        pltpu.sync_copy(nid.at[pl.ds(b, RC)], ibr)
        pltpu.async_copy(poi.at[ibr], ma, sem).wait()
        pltpu.sync_copy(ma, x0.at[pl.ds(b, RC), :])
        return 0

    n_a = jnp.where(wid == NW - 1, NRC - 2, NRC)
    lax.fori_loop(0, n_a, pass_a, 0)

    @pl.when(wid == NW - 1)
    def _():
        b = N - NTAIL_ROWS
        pltpu.sync_copy(nid.at[pl.ds(b, NTAIL_ROWS)], ibrt)
        pltpu.async_copy(
            poi.at[ibrt], ma.at[pl.ds(0, NTAIL_ROWS), :], sem).wait()
        pltpu.sync_copy(ma.at[pl.ds(0, NTAIL_ROWS), :],
                        x0.at[pl.ds(b, NTAIL_ROWS), :])

    _zero_rows(m0a, RC, D)
    _fill_1d(dbuf, RC, 0.0)
    _fill_1d(ones_v, EC, 1.0)
    lbase = s * TILE_ROWS

    def zf(j, _):
        lb = lbase + j * RC
        pltpu.sync_copy(m0a.at[pl.ds(0, RC), :], seg_sh.at[pl.ds(lb, RC), :])
        pltpu.sync_copy(dbuf, deg_sh.at[pl.ds(lb, RC)])
        return 0

    lax.fori_loop(0, NRC, zf, 0)
    plsc.subcore_barrier()

    ebase0 = s * EDGES_PER_TILE
    base = c * HALF
    ib0s, ib1s = (ib0a, ib0b), (ib1a, ib1b)
    locs, m0s = (loca, locb), (m0a, m0b)
    sis, sgs, sss, sds = (si0, si1), (sg0, sg1), (ss0, ss1), (sd0, sd1)

    def idx_issue(k, b):
        eb = ebase0 + k * EC
        pltpu.async_copy(dst_e.at[pl.ds(eb, EC)], ib0s[b], sis[b])
        pltpu.async_copy(dist_e.at[pl.ds(eb, EC)], ib1s[b], sis[b])

    def idx_drain(b):
        pltpu.make_async_copy(dst_e.at[pl.ds(0, EC)], ib0s[b], sis[b]).wait()
        pltpu.make_async_copy(dist_e.at[pl.ds(0, EC)], ib1s[b], sis[b]).wait()

    def g_drain(b):
        pltpu.make_async_copy(dembs.at[ib1s[b]], m0s[b], sgs[b]).wait()

    def s_drain(b):
        pltpu.make_async_copy(m0s[b], seg_sh.at[locs[b]], sss[b]).wait()

    def d_drain(b):
        pltpu.make_async_copy(ones_v, deg_sh.at[locs[b]], sds[b]).wait()

    idx_issue(0, 0)
    idx_drain(0)
    pltpu.async_copy(dembs.at[ib1s[0]], m0s[0], sgs[0])
    idx_issue(1, 1)

    def outer_b(g, _):
        for b in (0, 1):
            nb = 1 - b
            k = 2 * g + b

            @pl.when(k >= 1)
            def _():
                s_drain(nb)
                d_drain(nb)

            @pl.when(k + 1 < NECH)
            def _():
                idx_drain(nb)
                pltpu.async_copy(dembs.at[ib1s[nb]], m0s[nb], sgs[nb])

            _local_dst(ib0s[b], locs[b], base, EC)
            g_drain(b)

            @pl.when(k + 2 < NECH)
            def _():
                idx_issue(k + 2, b)

            pltpu.async_copy(m0s[b], seg_sh.at[locs[b]], sss[b], add=True)
            pltpu.async_copy(ones_v, deg_sh.at[locs[b]], sds[b], add=True)
        return 0

    lax.fori_loop(0, NECH // 2, outer_b, 0)
    s_drain(1)
    d_drain(1)

    et = ebase0 + NECH * EC
    pltpu.sync_copy(dst_e.at[pl.ds(et, ETAIL)], ib0t)
    pltpu.sync_copy(dist_e.at[pl.ds(et, ETAIL)], ib1t)
    _local_dst(ib0t, loct, base, ETAIL)
    pltpu.async_copy(
        dembs.at[ib1t], m0a.at[pl.ds(0, ETAIL), :], sem).wait()
    pltpu.sync_copy(m0a.at[pl.ds(0, ETAIL), :], seg_sh.at[loct], add=True)
    pltpu.sync_copy(ones_v.at[pl.ds(0, ETAIL)], deg_sh.at[loct], add=True)
    plsc.subcore_barrier()

    def wb(j, _):
        lb = lbase + j * RC
        g = base + lb
        pltpu.sync_copy(seg_sh.at[pl.ds(lb, RC), :], m0a.at[pl.ds(0, RC), :])
        pltpu.sync_copy(m0a.at[pl.ds(0, RC), :], segd.at[pl.ds(g, RC), :])
        pltpu.sync_copy(deg_sh.at[pl.ds(lb, RC)], dbuf)
        for i in range(RC // 16):
            dbuf[pl.ds(i * 16, 16)] = jnp.maximum(dbuf[pl.ds(i * 16, 16)], 1.0)
        pltpu.sync_copy(dbuf, deg.at[pl.ds(g, RC)])
        return 0

    lax.fori_loop(0, NRC, wb, 0)


def _layer_body(final, xl, segd, deg, src_e, dst_e, wb_arr,
                y,
                ib0a, ib0b, ib1a, ib1b, loca, locb,
                ib0t, ib1t, loct, m0a, m0b, m1,
                dbuf, rbuf, wv, sem,
                si0, si1, sg0, sg1, ss0, ss1,
                agg_sh):
    c = lax.axis_index("c")
    s = lax.axis_index("s")
    wid = c * NS + s
    base = c * HALF
    lbase = s * TILE_ROWS

    if final:
        pltpu.sync_copy(wb_arr, wv)

    def p1(j, _):
        lb = lbase + j * RC
        g = base + lb
        pltpu.sync_copy(segd.at[pl.ds(g, RC), :], m0a.at[pl.ds(0, RC), :])
        pltpu.sync_copy(m0a.at[pl.ds(0, RC), :], agg_sh.at[pl.ds(lb, RC), :])
        return 0

    lax.fori_loop(0, NRC, p1, 0)
    plsc.subcore_barrier()

    ebase0 = s * EDGES_PER_TILE
    ib0s, ib1s = (ib0a, ib0b), (ib1a, ib1b)
    locs, m0s = (loca, locb), (m0a, m0b)
    sis, sgs, sss = (si0, si1), (sg0, sg1), (ss0, ss1)

    def idx_issue(k, b):
        eb = ebase0 + k * EC
        pltpu.async_copy(src_e.at[pl.ds(eb, EC)], ib0s[b], sis[b])
        pltpu.async_copy(dst_e.at[pl.ds(eb, EC)], ib1s[b], sis[b])

    def idx_drain(b):
        pltpu.make_async_copy(src_e.at[pl.ds(0, EC)], ib0s[b], sis[b]).wait()
        pltpu.make_async_copy(dst_e.at[pl.ds(0, EC)], ib1s[b], sis[b]).wait()

    def g_drain(b):
        pltpu.make_async_copy(xl.at[ib0s[b]], m0s[b], sgs[b]).wait()

    def s_drain(b):
        pltpu.make_async_copy(m0s[b], agg_sh.at[locs[b]], sss[b]).wait()

    idx_issue(0, 0)
    idx_drain(0)
    pltpu.async_copy(xl.at[ib0s[0]], m0s[0], sgs[0])
    idx_issue(1, 1)

    def p2(g, _):
        for b in (0, 1):
            nb = 1 - b
            k = 2 * g + b

            @pl.when(k >= 1)
            def _():
                s_drain(nb)

            @pl.when(k + 1 < NECH)
            def _():
                idx_drain(nb)
                pltpu.async_copy(xl.at[ib0s[nb]], m0s[nb], sgs[nb])

            _local_dst(ib1s[b], locs[b], base, EC)
            g_drain(b)

            @pl.when(k + 2 < NECH)
            def _():
                idx_issue(k + 2, b)

            pltpu.async_copy(m0s[b], agg_sh.at[locs[b]], sss[b], add=True)
        return 0

    lax.fori_loop(0, NECH // 2, p2, 0)
    s_drain(1)

    et = ebase0 + NECH * EC
    pltpu.sync_copy(src_e.at[pl.ds(et, ETAIL)], ib0t)
    pltpu.sync_copy(dst_e.at[pl.ds(et, ETAIL)], ib1t)
    _local_dst(ib1t, loct, base, ETAIL)
    pltpu.async_copy(xl.at[ib0t], m0a.at[pl.ds(0, ETAIL), :], sem).wait()
    pltpu.sync_copy(m0a.at[pl.ds(0, ETAIL), :], agg_sh.at[loct], add=True)
    plsc.subcore_barrier()

    def p3(j, _):
        lb = lbase + j * RC
        g = base + lb
        pltpu.sync_copy(agg_sh.at[pl.ds(lb, RC), :], m0a.at[pl.ds(0, RC), :])
        pltpu.sync_copy(xl.at[pl.ds(g, RC), :], m1.at[pl.ds(0, RC), :])
        pltpu.sync_copy(deg.at[pl.ds(g, RC)], dbuf)
        for i in range(RC // 16):
            rbuf[pl.ds(i * 16, 16)] = 1.0 / dbuf[pl.ds(i * 16, 16)]

        if final:
            def row(r, _):
                rv = rbuf[pl.ds(r, 16)][0]
                acc = jnp.zeros((16,), jnp.float32)
                for f in range(D // 16):
                    yv = m1[r, pl.ds(f * 16, 16)] + m0a[r, pl.ds(f * 16, 16)] * rv
                    m0a[r, pl.ds(f * 16, 16)] = yv
                    acc = acc + yv * wv[pl.ds(f * 16, 16)]
                z = jnp.sum(acc)
                zb = lax.broadcast_in_dim(z, (16,), ()) + wv[pl.ds(D, 16)]
                gv = 1.0 / (1.0 + jnp.exp(-zb))
                for f in range(D // 16):
                    m0a[r, pl.ds(f * 16, 16)] = m0a[r, pl.ds(f * 16, 16)] * gv
                return 0
        else:
            def row(r, _):
                rv = rbuf[pl.ds(r, 16)][0]
                for f in range(D // 16):
                    m0a[r, pl.ds(f * 16, 16)] = (
                        m1[r, pl.ds(f * 16, 16)] + m0a[r, pl.ds(f * 16, 16)] * rv
                    )
                return 0

        lax.fori_loop(0, RC, row, 0)
        pltpu.sync_copy(m0a.at[pl.ds(0, RC), :], y.at[pl.ds(g, RC), :])
        return 0

    if final:
        n_j = jnp.where(wid == NW - 1, NRC - 2, NRC)
        lax.fori_loop(0, n_j, p3, 0)

        @pl.when(wid == NW - 1)
        def _():
            j = NRC - 2
            lb = lbase + j * RC
            g = base + lb
            pltpu.sync_copy(agg_sh.at[pl.ds(lb, RC), :],
                            m0a.at[pl.ds(0, RC), :])
            pltpu.sync_copy(xl.at[pl.ds(g, RC), :], m1.at[pl.ds(0, RC), :])
            pltpu.sync_copy(deg.at[pl.ds(g, RC)], dbuf)
            for i in range(RC // 16):
                rbuf[pl.ds(i * 16, 16)] = 1.0 / dbuf[pl.ds(i * 16, 16)]

            def row(r, _):
                rv = rbuf[pl.ds(r, 16)][0]
                acc = jnp.zeros((16,), jnp.float32)
                for f in range(D // 16):
                    yv = m1[r, pl.ds(f * 16, 16)] + m0a[r, pl.ds(f * 16, 16)] * rv
                    m0a[r, pl.ds(f * 16, 16)] = yv
                    acc = acc + yv * wv[pl.ds(f * 16, 16)]
                z = jnp.sum(acc)
                zb = lax.broadcast_in_dim(z, (16,), ()) + wv[pl.ds(D, 16)]
                gv = 1.0 / (1.0 + jnp.exp(-zb))
                for f in range(D // 16):
                    m0a[r, pl.ds(f * 16, 16)] = m0a[r, pl.ds(f * 16, 16)] * gv
                return 0

            lax.fori_loop(0, NTAIL_ROWS, row, 0)
            pltpu.sync_copy(m0a.at[pl.ds(0, NTAIL_ROWS), :],
                            y.at[pl.ds(g, NTAIL_ROWS), :])
    else:
        lax.fori_loop(0, NRC, p3, 0)


def _make_k1():
    return pl.kernel(
        _k1_body,
        out_type=(
            jax.ShapeDtypeStruct((NPAD, D), jnp.float32),
            jax.ShapeDtypeStruct((NPAD, D), jnp.float32),
            jax.ShapeDtypeStruct((NPAD,), jnp.float32),
        ),
        mesh=_mesh,
        scratch_types=[
            pltpu.VMEM((RC,), jnp.int32),
            pltpu.VMEM((NTAIL_ROWS,), jnp.int32),
            pltpu.VMEM((EC,), jnp.int32),
            pltpu.VMEM((EC,), jnp.int32),
            pltpu.VMEM((EC,), jnp.int32),
            pltpu.VMEM((EC,), jnp.int32),
            pltpu.VMEM((EC,), jnp.int32),
            pltpu.VMEM((EC,), jnp.int32),
            pltpu.VMEM((ETAIL,), jnp.int32),
            pltpu.VMEM((ETAIL,), jnp.int32),
            pltpu.VMEM((ETAIL,), jnp.int32),
            pltpu.VMEM((EC, D), jnp.float32),
            pltpu.VMEM((EC, D), jnp.float32),
            pltpu.VMEM((RC, D), jnp.float32),
            pltpu.VMEM((EC,), jnp.float32),
            pltpu.VMEM((RC,), jnp.float32),
            pltpu.SemaphoreType.DMA,
            pltpu.SemaphoreType.DMA,
            pltpu.SemaphoreType.DMA,
            pltpu.SemaphoreType.DMA,
            pltpu.SemaphoreType.DMA,
            pltpu.SemaphoreType.DMA,
            pltpu.SemaphoreType.DMA,
            pltpu.SemaphoreType.DMA,
            pltpu.SemaphoreType.DMA,
            pltpu.VMEM_SHARED((AGG_ROWS, D), jnp.float32),
            pltpu.VMEM_SHARED((AGG_ROWS,), jnp.float32),
        ],
        compiler_params=_params,
        name="mgdc_k1",
    )


def _make_layer(final):
    out_rows = N if final else NPAD
    return pl.kernel(
        functools.partial(_layer_body, final),
        out_type=jax.ShapeDtypeStruct((out_rows, D), jnp.float32),
        mesh=_mesh,
        scratch_types=[
            pltpu.VMEM((EC,), jnp.int32),
            pltpu.VMEM((EC,), jnp.int32),
            pltpu.VMEM((EC,), jnp.int32),
            pltpu.VMEM((EC,), jnp.int32),
            pltpu.VMEM((EC,), jnp.int32),
            pltpu.VMEM((EC,), jnp.int32),
            pltpu.VMEM((ETAIL,), jnp.int32),
            pltpu.VMEM((ETAIL,), jnp.int32),
            pltpu.VMEM((ETAIL,), jnp.int32),
            pltpu.VMEM((EC, D), jnp.float32),
            pltpu.VMEM((EC, D), jnp.float32),
            pltpu.VMEM((EC, D), jnp.float32),
            pltpu.VMEM((RC,), jnp.float32),
            pltpu.VMEM((RC + 16,), jnp.float32),
            pltpu.VMEM((D + 16,), jnp.float32),
            pltpu.SemaphoreType.DMA,
            pltpu.SemaphoreType.DMA,
            pltpu.SemaphoreType.DMA,
            pltpu.SemaphoreType.DMA,
            pltpu.SemaphoreType.DMA,
            pltpu.SemaphoreType.DMA,
            pltpu.SemaphoreType.DMA,
            pltpu.VMEM_SHARED((AGG_ROWS, D), jnp.float32),
        ],
        compiler_params=_params,
        name="mgdc_layer",
    )


def kernel(node_ids, edge_index, edge_dist, poi_table, delta_dis_embs,
           w_gate, b_gate):
    node_ids = node_ids.astype(jnp.int32)
    src = edge_index[0].astype(jnp.int32)
    dst = edge_index[1].astype(jnp.int32)
    dist = edge_dist.astype(jnp.int32)
    poi = poi_table.astype(jnp.float32)
    dembs = delta_dis_embs.astype(jnp.float32)

    wb_arr = jnp.concatenate(
        [w_gate.reshape(D).astype(jnp.float32),
         jnp.full((16,), b_gate.reshape(-1)[0], jnp.float32)]
    )

    k1 = _make_k1()
    layer = _make_layer(final=False)
    layer_final = _make_layer(final=True)

    x0, segd, deg = k1(node_ids, poi, dst, dist, dembs)
    x1 = layer(x0, segd, deg, src, dst, wb_arr)
    out = layer_final(x1, segd, deg, src, dst, wb_arr)
    return out
